# baseline (speedup 1.0000x reference)
import functools

import jax
import jax.numpy as jnp
from jax import lax
from jax.experimental import pallas as pl
from jax.experimental.pallas import tpu as pltpu
from jax.experimental.pallas import tpu_sc as plsc

HIDDEN = 128
SEQ = 200
L = 16
NSEG = HIDDEN // L
NW = 32
CHUNK = 100
NBUF = 4


@functools.lru_cache(maxsize=None)
def _make_sc_kernel(batch: int):
    n_tokens = batch * SEQ
    assert n_tokens % (NW * SEQ) == 0
    tok_per_w = n_tokens // NW
    chunks_per_w = tok_per_w // CHUNK
    assert chunks_per_w % NBUF == 0
    groups = chunks_per_w // NBUF

    mesh = plsc.VectorSubcoreMesh(core_axis_name="c", subcore_axis_name="s")

    @functools.partial(
        pl.kernel,
        mesh=mesh,
        compiler_params=pltpu.CompilerParams(needs_layout_passes=False,
                                             use_tc_tiling_on_sc=False),
        out_type=jax.ShapeDtypeStruct((n_tokens, HIDDEN), jnp.float32),
        scratch_types=[
            pltpu.VMEM((chunks_per_w, CHUNK), jnp.int32),
            pltpu.VMEM((tok_per_w + L,), jnp.float32),
            pltpu.VMEM((SEQ, HIDDEN), jnp.float32),
            pltpu.VMEM((2, HIDDEN), jnp.float32),
            pltpu.VMEM((HIDDEN,), jnp.float32),
            pltpu.VMEM((HIDDEN,), jnp.float32),
            pltpu.VMEM((CHUNK, HIDDEN), jnp.float32),
            pltpu.VMEM((CHUNK, HIDDEN), jnp.float32),
            pltpu.VMEM((CHUNK, HIDDEN), jnp.float32),
            pltpu.VMEM((CHUNK, HIDDEN), jnp.float32),
            pltpu.SemaphoreType.DMA,
            pltpu.SemaphoreType.DMA,
            pltpu.SemaphoreType.DMA,
            pltpu.SemaphoreType.DMA,
            pltpu.SemaphoreType.DMA,
            pltpu.SemaphoreType.DMA,
            pltpu.SemaphoreType.DMA,
            pltpu.SemaphoreType.DMA,
        ],
    )
    def sc_kernel(ids_hbm, ttf_hbm, word_hbm, pos_hbm, type_hbm, gamma_hbm,
                  beta_hbm, out_hbm, ids_v, ttf_v, post0_v, type_v, gamma_v,
                  beta_v, buf0, buf1, buf2, buf3, sem_g0, sem_g1, sem_g2,
                  sem_g3, sem_w0, sem_w1, sem_w2, sem_w3):
        bufs = [buf0, buf1, buf2, buf3]
        sg = [sem_g0, sem_g1, sem_g2, sem_g3]
        sw = [sem_w0, sem_w1, sem_w2, sem_w3]

        wid = lax.axis_index("s") * 2 + lax.axis_index("c")
        tok_base = wid * tok_per_w
        row_base = wid * chunks_per_w

        pltpu.sync_copy(pos_hbm.at[pl.ds(0, SEQ)], post0_v)
        pltpu.sync_copy(type_hbm, type_v)
        pltpu.sync_copy(gamma_hbm, gamma_v)
        pltpu.sync_copy(beta_hbm, beta_v)
        pltpu.sync_copy(ids_hbm.at[pl.ds(row_base, chunks_per_w)], ids_v)
        pltpu.sync_copy(ttf_hbm.at[pl.ds(tok_base, tok_per_w)],
                        ttf_v.at[pl.ds(0, tok_per_w)])

        g = [gamma_v[pl.ds(s * L, L)] for s in range(NSEG)]
        bt = [beta_v[pl.ds(s * L, L)] for s in range(NSEG)]
        t0 = [type_v[0, pl.ds(s * L, L)] for s in range(NSEG)]
        td = [type_v[1, pl.ds(s * L, L)] - t0[s] for s in range(NSEG)]

        e0 = (lax.iota(jnp.int32, L) == 0).astype(jnp.float32)

        def bcast0(v):
            return plsc.cumsum(v * e0)

        def bcast_total(v):
            return bcast0(jnp.flip(plsc.cumsum(v), 0))

        def fold_body(r, carry):
            for s in range(NSEG):
                ds = pl.ds(s * L, L)
                post0_v[r, ds] = post0_v[r, ds] + t0[s]
            return carry

        lax.fori_loop(0, SEQ, fold_body, 0, unroll=2)

        def start_gather(qq, b):
            pltpu.async_copy(word_hbm.at[pl.ds(qq * CHUNK, CHUNK)], bufs[b],
                             sg[b])

        def gather_wait(b):
            pltpu.make_async_copy(word_hbm.at[ids_v.at[0]], bufs[b],
                                  sg[b]).wait()

        def write_wait(b):
            pass

        for qq in range(NBUF - 1):
            start_gather(qq, qq)

        def group_body(grp, carry):
            for b in range(NBUF):
                q = grp * NBUF + b
                buf = bufs[b]
                pos_base = (b % 2) * CHUNK
                ttq_base = q * CHUNK

                gather_wait(b)

                def tok_body(i, tcarry):
                    ttf = bcast0(ttf_v[pl.ds(ttq_base + i, L)])
                    xs = []
                    acc_s = None
                    acc_q = None
                    for s in range(NSEG):
                        ds = pl.ds(s * L, L)
                        x = (buf[i, ds] + post0_v[pos_base + i, ds] +
                             ttf * td[s])
                        xs.append(x)
                        acc_s = x if acc_s is None else acc_s + x
                        acc_q = x * x if acc_q is None else acc_q + x * x
                    mean = bcast_total(acc_s) * (1.0 / HIDDEN)
                    sq = bcast_total(acc_q) * (1.0 / HIDDEN)
                    var = sq - mean * mean
                    xv = var + 1e-5
                    xh = 0.5 * xv
                    yi = 0x5F3759DF - (plsc.bitcast(xv, jnp.int32) >> 1)
                    y = plsc.bitcast(yi, jnp.float32)
                    for _ in range(2):
                        y = y * (1.5 - xh * y * y)
                    for s in range(NSEG):
                        buf[i, pl.ds(s * L, L)] = ((xs[s] - mean) * y *
                                                   g[s] + bt[s])
                    return tcarry

                lax.fori_loop(0, CHUNK, tok_body, 0, unroll=2)

                b3 = (b + NBUF - 1) % NBUF

                def refill():
                    start_gather(q + NBUF - 1, b3)

                if b == 0:
                    pl.when(grp >= 1)(lambda: write_wait(b3))
                    refill()
                else:
                    write_wait(b3)
                    pl.when(grp < groups - 1)(refill)

                def _dbg_writeback():
                    pltpu.async_copy(
                        buf, out_hbm.at[pl.ds(tok_base + q * CHUNK, CHUNK)],
                        sw[b])

                pl.when(q < 0)(_dbg_writeback)
            return carry

        lax.fori_loop(0, groups, group_body, 0)
        write_wait(NBUF - 1)

    return sc_kernel


def kernel(input_ids, token_type_ids, word_emb, pos_emb, type_emb, ln_gamma,
           ln_beta):
    batch, seq = input_ids.shape
    assert seq == SEQ
    ids = input_ids.astype(jnp.int32).reshape(-1, CHUNK)
    ttf = token_type_ids.astype(jnp.float32).reshape(-1)
    out = _make_sc_kernel(batch)(ids, ttf, word_emb, pos_emb, type_emb,
                                 ln_gamma, ln_beta)
    return out.reshape(batch, seq, HIDDEN)


# speedup vs baseline: 1.0977x; 1.0977x over previous
import functools

import jax
import jax.numpy as jnp
from jax import lax
from jax.experimental import pallas as pl
from jax.experimental.pallas import tpu as pltpu
from jax.experimental.pallas import tpu_sc as plsc

HIDDEN = 128
SEQ = 200
L = 16
NSEG = HIDDEN // L
NW = 32
CHUNK = 100
NBUF = 4


@functools.lru_cache(maxsize=None)
def _make_sc_kernel(batch: int):
    n_tokens = batch * SEQ
    assert n_tokens % (NW * SEQ) == 0
    tok_per_w = n_tokens // NW
    chunks_per_w = tok_per_w // CHUNK
    assert chunks_per_w % NBUF == 0
    groups = chunks_per_w // NBUF

    mesh = plsc.VectorSubcoreMesh(core_axis_name="c", subcore_axis_name="s")

    @functools.partial(
        pl.kernel,
        mesh=mesh,
        compiler_params=pltpu.CompilerParams(needs_layout_passes=False,
                                             use_tc_tiling_on_sc=False),
        out_type=jax.ShapeDtypeStruct((n_tokens, HIDDEN), jnp.float32),
        scratch_types=[
            pltpu.VMEM((chunks_per_w, CHUNK), jnp.int32),
            pltpu.VMEM((tok_per_w + L,), jnp.float32),
            pltpu.VMEM((SEQ, HIDDEN), jnp.float32),
            pltpu.VMEM((2, HIDDEN), jnp.float32),
            pltpu.VMEM((HIDDEN,), jnp.float32),
            pltpu.VMEM((HIDDEN,), jnp.float32),
            pltpu.VMEM((CHUNK, HIDDEN), jnp.float32),
            pltpu.VMEM((CHUNK, HIDDEN), jnp.float32),
            pltpu.VMEM((CHUNK, HIDDEN), jnp.float32),
            pltpu.VMEM((CHUNK, HIDDEN), jnp.float32),
            pltpu.SemaphoreType.DMA,
            pltpu.SemaphoreType.DMA,
            pltpu.SemaphoreType.DMA,
            pltpu.SemaphoreType.DMA,
            pltpu.SemaphoreType.DMA,
            pltpu.SemaphoreType.DMA,
            pltpu.SemaphoreType.DMA,
            pltpu.SemaphoreType.DMA,
        ],
    )
    def sc_kernel(ids_hbm, ttf_hbm, word_hbm, pos_hbm, type_hbm, gamma_hbm,
                  beta_hbm, out_hbm, ids_v, ttf_v, post0_v, type_v, gamma_v,
                  beta_v, buf0, buf1, buf2, buf3, sem_g0, sem_g1, sem_g2,
                  sem_g3, sem_w0, sem_w1, sem_w2, sem_w3):
        bufs = [buf0, buf1, buf2, buf3]
        sg = [sem_g0, sem_g1, sem_g2, sem_g3]
        sw = [sem_w0, sem_w1, sem_w2, sem_w3]

        wid = lax.axis_index("s") * 2 + lax.axis_index("c")
        tok_base = wid * tok_per_w
        row_base = wid * chunks_per_w

        pltpu.sync_copy(pos_hbm.at[pl.ds(0, SEQ)], post0_v)
        pltpu.sync_copy(type_hbm, type_v)
        pltpu.sync_copy(gamma_hbm, gamma_v)
        pltpu.sync_copy(beta_hbm, beta_v)
        pltpu.sync_copy(ids_hbm.at[pl.ds(row_base, chunks_per_w)], ids_v)
        pltpu.sync_copy(ttf_hbm.at[pl.ds(tok_base, tok_per_w)],
                        ttf_v.at[pl.ds(0, tok_per_w)])

        g = [gamma_v[pl.ds(s * L, L)] for s in range(NSEG)]
        bt = [beta_v[pl.ds(s * L, L)] for s in range(NSEG)]
        t0 = [type_v[0, pl.ds(s * L, L)] for s in range(NSEG)]
        td = [type_v[1, pl.ds(s * L, L)] - t0[s] for s in range(NSEG)]

        e0 = (lax.iota(jnp.int32, L) == 0).astype(jnp.float32)

        def bcast0(v):
            return plsc.cumsum(v * e0)

        def bcast_total(v):
            return bcast0(jnp.flip(plsc.cumsum(v), 0))

        def fold_body(r, carry):
            for s in range(NSEG):
                ds = pl.ds(s * L, L)
                post0_v[r, ds] = post0_v[r, ds] + t0[s]
            return carry

        lax.fori_loop(0, SEQ, fold_body, 0, unroll=2)

        def start_gather(qq, b):
            pltpu.async_copy(word_hbm.at[ids_v.at[qq]], bufs[b], sg[b])

        def gather_wait(b):
            pltpu.make_async_copy(word_hbm.at[ids_v.at[0]], bufs[b],
                                  sg[b]).wait()

        def write_wait(b):
            pltpu.make_async_copy(bufs[b], out_hbm.at[pl.ds(0, CHUNK)],
                                  sw[b]).wait()

        for qq in range(NBUF - 1):
            start_gather(qq, qq)

        def group_body(grp, carry):
            for b in range(NBUF):
                q = grp * NBUF + b
                buf = bufs[b]
                pos_base = (b % 2) * CHUNK
                ttq_base = q * CHUNK

                gather_wait(b)

                def stats(i):
                    ttf = bcast0(ttf_v[pl.ds(ttq_base + i, L)])
                    acc_s = None
                    acc_q = None
                    for s in range(NSEG):
                        ds = pl.ds(s * L, L)
                        x = (buf[i, ds] + post0_v[pos_base + i, ds] +
                             ttf * td[s])
                        buf[i, ds] = x
                        acc_s = x if acc_s is None else acc_s + x
                        acc_q = x * x if acc_q is None else acc_q + x * x
                    return acc_s, acc_q

                def finish(j, acc_s, acc_q):
                    mean = bcast_total(acc_s) * (1.0 / HIDDEN)
                    sq = bcast_total(acc_q) * (1.0 / HIDDEN)
                    var = sq - mean * mean
                    xv = var + 1e-5
                    xh = 0.5 * xv
                    yi = 0x5F3759DF - (plsc.bitcast(xv, jnp.int32) >> 1)
                    y = plsc.bitcast(yi, jnp.float32)
                    for _ in range(2):
                        y = y * (1.5 - xh * y * y)
                    for s in range(NSEG):
                        ds = pl.ds(s * L, L)
                        buf[j, ds] = (buf[j, ds] - mean) * y * g[s] + bt[s]

                def tok_body(i, tcarry):
                    pa, pq = tcarry
                    na, nq = stats(i)
                    finish(i - 1, pa, pq)
                    return na, nq

                first = stats(0)
                last = lax.fori_loop(1, CHUNK, tok_body, first, unroll=2)
                finish(CHUNK - 1, *last)

                b3 = (b + NBUF - 1) % NBUF

                def refill():
                    start_gather(q + NBUF - 1, b3)

                if b == 0:
                    pl.when(grp >= 1)(lambda: write_wait(b3))
                    refill()
                else:
                    write_wait(b3)
                    pl.when(grp < groups - 1)(refill)

                pltpu.async_copy(
                    buf, out_hbm.at[pl.ds(tok_base + q * CHUNK, CHUNK)],
                    sw[b])
            return carry

        lax.fori_loop(0, groups, group_body, 0)
        write_wait(NBUF - 1)

    return sc_kernel


def kernel(input_ids, token_type_ids, word_emb, pos_emb, type_emb, ln_gamma,
           ln_beta):
    batch, seq = input_ids.shape
    assert seq == SEQ
    ids = input_ids.astype(jnp.int32).reshape(-1, CHUNK)
    ttf = token_type_ids.astype(jnp.float32).reshape(-1)
    out = _make_sc_kernel(batch)(ids, ttf, word_emb, pos_emb, type_emb,
                                 ln_gamma, ln_beta)
    return out.reshape(batch, seq, HIDDEN)


# speedup vs baseline: 1.8277x; 1.6651x over previous
import functools

import jax
import jax.numpy as jnp
from jax import lax
from jax.experimental import pallas as pl
from jax.experimental.pallas import tpu as pltpu
from jax.experimental.pallas import tpu_sc as plsc

HIDDEN = 128
SEQ = 200
L = 16
NSEG = HIDDEN // L
NW = 32
CHUNK = 100
NBUF = 4


@functools.lru_cache(maxsize=None)
def _make_sc_kernel(batch: int):
    n_tokens = batch * SEQ
    assert n_tokens % (NW * SEQ) == 0
    tok_per_w = n_tokens // NW
    chunks_per_w = tok_per_w // CHUNK
    assert chunks_per_w % NBUF == 0
    groups = chunks_per_w // NBUF

    mesh = plsc.VectorSubcoreMesh(core_axis_name="c", subcore_axis_name="s")

    @functools.partial(
        pl.kernel,
        mesh=mesh,
        compiler_params=pltpu.CompilerParams(needs_layout_passes=False,
                                             use_tc_tiling_on_sc=False),
        out_type=jax.ShapeDtypeStruct((n_tokens, HIDDEN), jnp.float32),
        scratch_types=[
            pltpu.VMEM((chunks_per_w, CHUNK), jnp.int32),
            pltpu.VMEM((tok_per_w + L,), jnp.float32),
            pltpu.VMEM((SEQ, HIDDEN), jnp.float32),
            pltpu.VMEM((2, HIDDEN), jnp.float32),
            pltpu.VMEM((HIDDEN,), jnp.float32),
            pltpu.VMEM((HIDDEN,), jnp.float32),
            pltpu.VMEM((CHUNK, HIDDEN), jnp.float32),
            pltpu.VMEM((CHUNK, HIDDEN), jnp.float32),
            pltpu.VMEM((CHUNK, HIDDEN), jnp.float32),
            pltpu.VMEM((CHUNK, HIDDEN), jnp.float32),
            pltpu.SemaphoreType.DMA,
            pltpu.SemaphoreType.DMA,
            pltpu.SemaphoreType.DMA,
            pltpu.SemaphoreType.DMA,
            pltpu.SemaphoreType.DMA,
            pltpu.SemaphoreType.DMA,
            pltpu.SemaphoreType.DMA,
            pltpu.SemaphoreType.DMA,
        ],
    )
    def sc_kernel(ids_hbm, ttf_hbm, word_hbm, pos_hbm, type_hbm, gamma_hbm,
                  beta_hbm, out_hbm, ids_v, ttf_v, post0_v, type_v, gamma_v,
                  beta_v, buf0, buf1, buf2, buf3, sem_g0, sem_g1, sem_g2,
                  sem_g3, sem_w0, sem_w1, sem_w2, sem_w3):
        bufs = [buf0, buf1, buf2, buf3]
        sg = [sem_g0, sem_g1, sem_g2, sem_g3]
        sw = [sem_w0, sem_w1, sem_w2, sem_w3]

        wid = lax.axis_index("s") * 2 + lax.axis_index("c")
        tok_base = wid * tok_per_w
        row_base = wid * chunks_per_w

        pltpu.sync_copy(pos_hbm.at[pl.ds(0, SEQ)], post0_v)
        pltpu.sync_copy(type_hbm, type_v)
        pltpu.sync_copy(gamma_hbm, gamma_v)
        pltpu.sync_copy(beta_hbm, beta_v)
        pltpu.sync_copy(ids_hbm.at[pl.ds(row_base, chunks_per_w)], ids_v)
        pltpu.sync_copy(ttf_hbm.at[pl.ds(tok_base, tok_per_w)],
                        ttf_v.at[pl.ds(0, tok_per_w)])

        g = [gamma_v[pl.ds(s * L, L)] for s in range(NSEG)]
        bt = [beta_v[pl.ds(s * L, L)] for s in range(NSEG)]
        t0 = [type_v[0, pl.ds(s * L, L)] for s in range(NSEG)]
        td = [type_v[1, pl.ds(s * L, L)] - t0[s] for s in range(NSEG)]

        e0 = (lax.iota(jnp.int32, L) == 0).astype(jnp.float32)

        def bcast0(v):
            return plsc.cumsum(v * e0)

        def bcast_total(v):
            return bcast0(jnp.flip(plsc.cumsum(v), 0))

        def fold_body(r, carry):
            for s in range(NSEG):
                ds = pl.ds(s * L, L)
                post0_v[r, ds] = post0_v[r, ds] + t0[s]
            return carry

        lax.fori_loop(0, SEQ, fold_body, 0, unroll=2)

        def start_gather(qq, b):
            pltpu.async_copy(word_hbm.at[ids_v.at[qq]], bufs[b], sg[b])

        def gather_wait(b):
            pltpu.make_async_copy(word_hbm.at[ids_v.at[0]], bufs[b],
                                  sg[b]).wait()

        def write_wait(b):
            pltpu.make_async_copy(bufs[b], out_hbm.at[pl.ds(0, CHUNK)],
                                  sw[b]).wait()

        for qq in range(NBUF - 1):
            start_gather(qq, qq)

        def group_body(grp, carry):
            for b in range(NBUF):
                q = grp * NBUF + b
                buf = bufs[b]
                pos_base = (b % 2) * CHUNK
                ttq_base = q * CHUNK

                gather_wait(b)

                def stats(i):
                    ttf = bcast0(ttf_v[pl.ds(ttq_base + i, L)])
                    acc_s = None
                    acc_q = None
                    for s in range(NSEG):
                        ds = pl.ds(s * L, L)
                        x = (buf[i, ds] + post0_v[pos_base + i, ds] +
                             ttf * td[s])
                        buf[i, ds] = x
                        acc_s = x if acc_s is None else acc_s + x
                        acc_q = x * x if acc_q is None else acc_q + x * x
                    return acc_s, acc_q

                def finish(j, acc_s, acc_q):
                    mean = bcast_total(acc_s) * (1.0 / HIDDEN)
                    sq = bcast_total(acc_q) * (1.0 / HIDDEN)
                    var = sq - mean * mean
                    xv = var + 1e-5
                    xh = 0.5 * xv
                    yi = 0x5F3759DF - (plsc.bitcast(xv, jnp.int32) >> 1)
                    y = plsc.bitcast(yi, jnp.float32)
                    for _ in range(2):
                        y = y * (1.5 - xh * y * y)
                    for s in range(NSEG):
                        ds = pl.ds(s * L, L)
                        buf[j, ds] = (buf[j, ds] - mean) * y * g[s] + bt[s]

                def tok_body(i, tcarry):
                    pa, pq = tcarry
                    na, nq = stats(i)
                    finish(i - 1, pa, pq)
                    return na, nq

                first = stats(0)
                last = lax.fori_loop(1, CHUNK, tok_body, first, unroll=1)
                finish(CHUNK - 1, *last)

                b3 = (b + NBUF - 1) % NBUF

                def refill():
                    start_gather(q + NBUF - 1, b3)

                if b == 0:
                    pl.when(grp >= 1)(lambda: write_wait(b3))
                    refill()
                else:
                    write_wait(b3)
                    pl.when(grp < groups - 1)(refill)

                pltpu.async_copy(
                    buf, out_hbm.at[pl.ds(tok_base + q * CHUNK, CHUNK)],
                    sw[b])
            return carry

        lax.fori_loop(0, groups, group_body, 0)
        write_wait(NBUF - 1)

    return sc_kernel


def kernel(input_ids, token_type_ids, word_emb, pos_emb, type_emb, ln_gamma,
           ln_beta):
    batch, seq = input_ids.shape
    assert seq == SEQ
    ids = input_ids.astype(jnp.int32).reshape(-1, CHUNK)
    ttf = token_type_ids.astype(jnp.float32).reshape(-1)
    out = _make_sc_kernel(batch)(ids, ttf, word_emb, pos_emb, type_emb,
                                 ln_gamma, ln_beta)
    return out.reshape(batch, seq, HIDDEN)


# speedup vs baseline: 1.8876x; 1.0328x over previous
import functools

import jax
import jax.numpy as jnp
from jax import lax
from jax.experimental import pallas as pl
from jax.experimental.pallas import tpu as pltpu
from jax.experimental.pallas import tpu_sc as plsc

HIDDEN = 128
SEQ = 200
L = 16
NSEG = HIDDEN // L
NW = 32
CHUNK = 100
NBUF = 4


@functools.lru_cache(maxsize=None)
def _make_sc_kernel(batch: int):
    n_tokens = batch * SEQ
    assert n_tokens % (NW * SEQ) == 0
    tok_per_w = n_tokens // NW
    chunks_per_w = tok_per_w // CHUNK
    assert chunks_per_w % NBUF == 0
    groups = chunks_per_w // NBUF

    mesh = plsc.VectorSubcoreMesh(core_axis_name="c", subcore_axis_name="s")

    @functools.partial(
        pl.kernel,
        mesh=mesh,
        compiler_params=pltpu.CompilerParams(needs_layout_passes=False,
                                             use_tc_tiling_on_sc=False),
        out_type=jax.ShapeDtypeStruct((n_tokens, HIDDEN), jnp.float32),
        scratch_types=[
            pltpu.VMEM((chunks_per_w, CHUNK), jnp.int32),
            pltpu.VMEM((tok_per_w + L,), jnp.float32),
            pltpu.VMEM((SEQ, HIDDEN), jnp.float32),
            pltpu.VMEM((2, HIDDEN), jnp.float32),
            pltpu.VMEM((HIDDEN,), jnp.float32),
            pltpu.VMEM((HIDDEN,), jnp.float32),
            pltpu.VMEM((CHUNK, HIDDEN), jnp.float32),
            pltpu.VMEM((CHUNK, HIDDEN), jnp.float32),
            pltpu.VMEM((CHUNK, HIDDEN), jnp.float32),
            pltpu.VMEM((CHUNK, HIDDEN), jnp.float32),
            pltpu.SemaphoreType.DMA,
            pltpu.SemaphoreType.DMA,
            pltpu.SemaphoreType.DMA,
            pltpu.SemaphoreType.DMA,
            pltpu.SemaphoreType.DMA,
            pltpu.SemaphoreType.DMA,
            pltpu.SemaphoreType.DMA,
            pltpu.SemaphoreType.DMA,
        ],
    )
    def sc_kernel(ids_hbm, ttf_hbm, word_hbm, pos_hbm, type_hbm, gamma_hbm,
                  beta_hbm, out_hbm, ids_v, ttf_v, post0_v, type_v, gamma_v,
                  beta_v, buf0, buf1, buf2, buf3, sem_g0, sem_g1, sem_g2,
                  sem_g3, sem_w0, sem_w1, sem_w2, sem_w3):
        bufs = [buf0, buf1, buf2, buf3]
        sg = [sem_g0, sem_g1, sem_g2, sem_g3]
        sw = [sem_w0, sem_w1, sem_w2, sem_w3]

        wid = lax.axis_index("s") * 2 + lax.axis_index("c")
        tok_base = wid * tok_per_w
        row_base = wid * chunks_per_w

        pltpu.sync_copy(pos_hbm.at[pl.ds(0, SEQ)], post0_v)
        pltpu.sync_copy(type_hbm, type_v)
        pltpu.sync_copy(gamma_hbm, gamma_v)
        pltpu.sync_copy(beta_hbm, beta_v)
        pltpu.sync_copy(ids_hbm.at[pl.ds(row_base, chunks_per_w)], ids_v)
        pltpu.sync_copy(ttf_hbm.at[pl.ds(tok_base, tok_per_w)],
                        ttf_v.at[pl.ds(0, tok_per_w)])

        g = [gamma_v[pl.ds(s * L, L)] for s in range(NSEG)]
        bt = [beta_v[pl.ds(s * L, L)] for s in range(NSEG)]
        t0 = [type_v[0, pl.ds(s * L, L)] for s in range(NSEG)]
        td = [type_v[1, pl.ds(s * L, L)] - t0[s] for s in range(NSEG)]

        e0 = (lax.iota(jnp.int32, L) == 0).astype(jnp.float32)

        def bcast0(v):
            return plsc.cumsum(v * e0)

        def bcast_total(v):
            return bcast0(jnp.flip(plsc.cumsum(v), 0))

        def fold_body(r, carry):
            for s in range(NSEG):
                ds = pl.ds(s * L, L)
                post0_v[r, ds] = post0_v[r, ds] + t0[s]
            return carry

        lax.fori_loop(0, SEQ, fold_body, 0, unroll=2)

        def start_gather(qq, b):
            pltpu.async_copy(word_hbm.at[ids_v.at[qq]], bufs[b], sg[b])

        def gather_wait(b):
            pltpu.make_async_copy(word_hbm.at[ids_v.at[0]], bufs[b],
                                  sg[b]).wait()

        def write_wait(b):
            pltpu.make_async_copy(bufs[b], out_hbm.at[pl.ds(0, CHUNK)],
                                  sw[b]).wait()

        for qq in range(NBUF - 1):
            start_gather(qq, qq)

        def group_body(grp, carry):
            for b in range(NBUF):
                q = grp * NBUF + b
                buf = bufs[b]
                pos_base = (b % 2) * CHUNK
                ttq_base = q * CHUNK

                gather_wait(b)

                def stats(i):
                    ttf = bcast0(ttf_v[pl.ds(ttq_base + i, L)])
                    acc_s = None
                    acc_q = None
                    for s in range(NSEG):
                        ds = pl.ds(s * L, L)
                        x = (buf[i, ds] + post0_v[pos_base + i, ds] +
                             ttf * td[s])
                        buf[i, ds] = x
                        acc_s = x if acc_s is None else acc_s + x
                        acc_q = x * x if acc_q is None else acc_q + x * x
                    return acc_s, acc_q

                def finish(j, acc_s, acc_q):
                    mean = bcast_total(acc_s) * (1.0 / HIDDEN)
                    sq = bcast_total(acc_q) * (1.0 / HIDDEN)
                    var = sq - mean * mean
                    xv = var + 1e-5
                    xh = 0.5 * xv
                    yi = 0x5F3759DF - (plsc.bitcast(xv, jnp.int32) >> 1)
                    y = plsc.bitcast(yi, jnp.float32)
                    for _ in range(1):
                        y = y * (1.5 - xh * y * y)
                    for s in range(NSEG):
                        ds = pl.ds(s * L, L)
                        buf[j, ds] = (buf[j, ds] - mean) * y * g[s] + bt[s]

                def tok_body(i, tcarry):
                    pa, pq = tcarry
                    na, nq = stats(i)
                    finish(i - 1, pa, pq)
                    return na, nq

                first = stats(0)
                last = lax.fori_loop(1, CHUNK, tok_body, first, unroll=1)
                finish(CHUNK - 1, *last)

                b3 = (b + NBUF - 1) % NBUF

                def refill():
                    start_gather(q + NBUF - 1, b3)

                if b == 0:
                    pl.when(grp >= 1)(lambda: write_wait(b3))
                    refill()
                else:
                    write_wait(b3)
                    pl.when(grp < groups - 1)(refill)

                pltpu.async_copy(
                    buf, out_hbm.at[pl.ds(tok_base + q * CHUNK, CHUNK)],
                    sw[b])
            return carry

        lax.fori_loop(0, groups, group_body, 0)
        write_wait(NBUF - 1)

    return sc_kernel


def kernel(input_ids, token_type_ids, word_emb, pos_emb, type_emb, ln_gamma,
           ln_beta):
    batch, seq = input_ids.shape
    assert seq == SEQ
    ids = input_ids.astype(jnp.int32).reshape(-1, CHUNK)
    ttf = token_type_ids.astype(jnp.float32).reshape(-1)
    out = _make_sc_kernel(batch)(ids, ttf, word_emb, pos_emb, type_emb,
                                 ln_gamma, ln_beta)
    return out.reshape(batch, seq, HIDDEN)
